# baseline (device time: 25661 ns/iter reference)
import jax
import jax.numpy as jnp
from jax import lax
from jax.experimental import pallas as pl
from jax.experimental.pallas import tpu as pltpu

N_CHUNKS = 8


def kernel(x):
    m, n = x.shape
    half_n = n // 2
    half_m = m // 2
    ch = half_m // N_CHUNKS

    def body(
        x_ref, out_ref, hbm_ref, stage_ref,
        sx_sems, rx_sems, sy_sems, ry_sems, lc_sems,
    ):
        my_x = lax.axis_index("x")
        my_y = lax.axis_index("y")
        my_z = lax.axis_index("z")
        px = 1 - my_x
        s = my_y % 2
        py = my_y ^ 1

        barrier_sem = pltpu.get_barrier_semaphore()
        for dev in [(px, my_y, my_z), (my_x, py, my_z)]:
            pl.semaphore_signal(
                barrier_sem, inc=1,
                device_id=dev, device_id_type=pl.DeviceIdType.MESH,
            )
        pl.semaphore_wait(barrier_sem, 2)

        x_rdmas = []
        for c in range(N_CHUNKS):
            stage_ref[pl.ds(c * ch, ch), :] = x_ref[
                pl.ds(s * half_m + c * ch, ch), pl.ds(px * half_n, half_n)
            ].astype(jnp.bfloat16)
            rdma = pltpu.make_async_remote_copy(
                src_ref=stage_ref.at[pl.ds(c * ch, ch), :],
                dst_ref=out_ref.at[
                    pl.ds(my_x * m + s * half_m + c * ch, ch), :
                ],
                send_sem=sx_sems.at[c],
                recv_sem=rx_sems.at[c],
                device_id=(px, my_y, my_z),
                device_id_type=pl.DeviceIdType.MESH,
            )
            rdma.start()
            x_rdmas.append(rdma)

        out_ref[pl.ds(my_x * m, m), :] = x_ref[
            :, pl.ds(my_x * half_n, half_n)
        ].astype(jnp.bfloat16)

        y_rdmas = []
        for c in range(N_CHUNKS):
            x_rdmas[c].wait_recv()
            rows = pl.ds(px * m + s * half_m + c * ch, ch)
            fwd = pltpu.make_async_remote_copy(
                src_ref=out_ref.at[rows, :],
                dst_ref=hbm_ref.at[pl.ds(c * ch, ch), :],
                send_sem=sy_sems.at[c],
                recv_sem=ry_sems.at[c],
                device_id=(my_x, py, my_z),
                device_id_type=pl.DeviceIdType.MESH,
            )
            fwd.start()
            y_rdmas.append(fwd)

        local_copies = []
        for c in range(N_CHUNKS):
            y_rdmas[c].wait_recv()
            cp = pltpu.make_async_copy(
                hbm_ref.at[pl.ds(c * ch, ch), :],
                out_ref.at[pl.ds(px * m + (1 - s) * half_m + c * ch, ch), :],
                lc_sems.at[c],
            )
            cp.start()
            local_copies.append(cp)

        for c in range(N_CHUNKS):
            local_copies[c].wait()
            x_rdmas[c].wait_send()
            y_rdmas[c].wait_send()

    out, _ = pl.pallas_call(
        body,
        out_shape=(
            jax.ShapeDtypeStruct((2 * m, half_n), jnp.bfloat16),
            jax.ShapeDtypeStruct((half_m, half_n), jnp.bfloat16),
        ),
        in_specs=[pl.BlockSpec(memory_space=pltpu.VMEM)],
        out_specs=(
            pl.BlockSpec(memory_space=pltpu.VMEM),
            pl.BlockSpec(memory_space=pltpu.MemorySpace.HBM),
        ),
        scratch_shapes=[
            pltpu.VMEM((half_m, half_n), jnp.bfloat16),
            pltpu.SemaphoreType.DMA((N_CHUNKS,)),
            pltpu.SemaphoreType.DMA((N_CHUNKS,)),
            pltpu.SemaphoreType.DMA((N_CHUNKS,)),
            pltpu.SemaphoreType.DMA((N_CHUNKS,)),
            pltpu.SemaphoreType.DMA((N_CHUNKS,)),
        ],
        compiler_params=pltpu.CompilerParams(collective_id=0),
    )(x)
    return out


# device time: 24369 ns/iter; 1.0530x vs baseline; 1.0530x over previous
import jax
import jax.numpy as jnp
from jax import lax
from jax.experimental import pallas as pl
from jax.experimental.pallas import tpu as pltpu

N_CHUNKS = 8


def kernel(x):
    m, n = x.shape
    half_n = n // 2
    half_m = m // 2
    ch = half_m // N_CHUNKS

    def body(x_ref, out_ref, stage_ref, sx_sems, rx_sems, sy_sems, ry_sems):
        my_x = lax.axis_index("x")
        my_y = lax.axis_index("y")
        my_z = lax.axis_index("z")
        px = 1 - my_x
        s = my_y % 2
        py = my_y ^ 1

        barrier_sem = pltpu.get_barrier_semaphore()
        for dev in [(px, my_y, my_z), (my_x, py, my_z)]:
            pl.semaphore_signal(
                barrier_sem, inc=1,
                device_id=dev, device_id_type=pl.DeviceIdType.MESH,
            )

        x_rdmas = []
        for c in range(N_CHUNKS):
            stage_ref[pl.ds(c * ch, ch), :] = x_ref[
                pl.ds(s * half_m + c * ch, ch), pl.ds(px * half_n, half_n)
            ].astype(jnp.bfloat16)
            if c == 0:
                pl.semaphore_wait(barrier_sem, 2)
            rdma = pltpu.make_async_remote_copy(
                src_ref=stage_ref.at[pl.ds(c * ch, ch), :],
                dst_ref=out_ref.at[
                    pl.ds(my_x * m + s * half_m + c * ch, ch), :
                ],
                send_sem=sx_sems.at[c],
                recv_sem=rx_sems.at[c],
                device_id=(px, my_y, my_z),
                device_id_type=pl.DeviceIdType.MESH,
            )
            rdma.start()
            x_rdmas.append(rdma)

        out_ref[pl.ds(my_x * m, m), :] = x_ref[
            :, pl.ds(my_x * half_n, half_n)
        ].astype(jnp.bfloat16)

        y_rdmas = []
        for c in range(N_CHUNKS):
            x_rdmas[c].wait_recv()
            rows = pl.ds(px * m + s * half_m + c * ch, ch)
            fwd = pltpu.make_async_remote_copy(
                src_ref=out_ref.at[rows, :],
                dst_ref=out_ref.at[rows, :],
                send_sem=sy_sems.at[c],
                recv_sem=ry_sems.at[c],
                device_id=(my_x, py, my_z),
                device_id_type=pl.DeviceIdType.MESH,
            )
            fwd.start()
            y_rdmas.append(fwd)

        for c in range(N_CHUNKS):
            y_rdmas[c].wait_recv()
        for c in range(N_CHUNKS):
            x_rdmas[c].wait_send()
            y_rdmas[c].wait_send()

    return pl.pallas_call(
        body,
        out_shape=jax.ShapeDtypeStruct((2 * m, half_n), jnp.bfloat16),
        in_specs=[pl.BlockSpec(memory_space=pltpu.VMEM)],
        out_specs=pl.BlockSpec(memory_space=pltpu.VMEM),
        scratch_shapes=[
            pltpu.VMEM((half_m, half_n), jnp.bfloat16),
            pltpu.SemaphoreType.DMA((N_CHUNKS,)),
            pltpu.SemaphoreType.DMA((N_CHUNKS,)),
            pltpu.SemaphoreType.DMA((N_CHUNKS,)),
            pltpu.SemaphoreType.DMA((N_CHUNKS,)),
        ],
        compiler_params=pltpu.CompilerParams(collective_id=0),
    )(x)


# device time: 23972 ns/iter; 1.0705x vs baseline; 1.0166x over previous
import jax
import jax.numpy as jnp
from jax import lax
from jax.experimental import pallas as pl
from jax.experimental.pallas import tpu as pltpu

N_CHUNKS = 16


def kernel(x):
    m, n = x.shape
    half_n = n // 2
    half_m = m // 2
    ch = half_m // N_CHUNKS

    def body(x_ref, out_ref, stage_ref, sx_sems, rx_sems, sy_sems, ry_sems):
        my_x = lax.axis_index("x")
        my_y = lax.axis_index("y")
        my_z = lax.axis_index("z")
        px = 1 - my_x
        s = my_y % 2
        py = my_y ^ 1

        barrier_sem = pltpu.get_barrier_semaphore()
        for dev in [(px, my_y, my_z), (my_x, py, my_z)]:
            pl.semaphore_signal(
                barrier_sem, inc=1,
                device_id=dev, device_id_type=pl.DeviceIdType.MESH,
            )

        x_rdmas = []
        for c in range(N_CHUNKS):
            stage_ref[pl.ds(c * ch, ch), :] = x_ref[
                pl.ds(s * half_m + c * ch, ch), pl.ds(px * half_n, half_n)
            ].astype(jnp.bfloat16)
            if c == 0:
                pl.semaphore_wait(barrier_sem, 2)
            rdma = pltpu.make_async_remote_copy(
                src_ref=stage_ref.at[pl.ds(c * ch, ch), :],
                dst_ref=out_ref.at[
                    pl.ds(my_x * m + s * half_m + c * ch, ch), :
                ],
                send_sem=sx_sems.at[c],
                recv_sem=rx_sems.at[c],
                device_id=(px, my_y, my_z),
                device_id_type=pl.DeviceIdType.MESH,
            )
            rdma.start()
            x_rdmas.append(rdma)

        out_ref[pl.ds(my_x * m, m), :] = x_ref[
            :, pl.ds(my_x * half_n, half_n)
        ].astype(jnp.bfloat16)

        y_rdmas = []
        for c in range(N_CHUNKS):
            x_rdmas[c].wait_recv()
            rows = pl.ds(px * m + s * half_m + c * ch, ch)
            fwd = pltpu.make_async_remote_copy(
                src_ref=out_ref.at[rows, :],
                dst_ref=out_ref.at[rows, :],
                send_sem=sy_sems.at[c],
                recv_sem=ry_sems.at[c],
                device_id=(my_x, py, my_z),
                device_id_type=pl.DeviceIdType.MESH,
            )
            fwd.start()
            y_rdmas.append(fwd)

        for c in range(N_CHUNKS):
            y_rdmas[c].wait_recv()
        for c in range(N_CHUNKS):
            x_rdmas[c].wait_send()
            y_rdmas[c].wait_send()

    return pl.pallas_call(
        body,
        out_shape=jax.ShapeDtypeStruct((2 * m, half_n), jnp.bfloat16),
        in_specs=[pl.BlockSpec(memory_space=pltpu.VMEM)],
        out_specs=pl.BlockSpec(memory_space=pltpu.VMEM),
        scratch_shapes=[
            pltpu.VMEM((half_m, half_n), jnp.bfloat16),
            pltpu.SemaphoreType.DMA((N_CHUNKS,)),
            pltpu.SemaphoreType.DMA((N_CHUNKS,)),
            pltpu.SemaphoreType.DMA((N_CHUNKS,)),
            pltpu.SemaphoreType.DMA((N_CHUNKS,)),
        ],
        compiler_params=pltpu.CompilerParams(collective_id=0),
    )(x)
